# baseline (speedup 1.0000x reference)
import jax
import jax.numpy as jnp
from jax import lax
from jax.experimental import pallas as pl
from jax.experimental.pallas import tpu as pltpu
from jax.experimental.pallas import tpu_sc as plsc

N_TOTAL = 10000
N_PAD = 10240
B1 = 400
NB1 = N_TOTAL // B1
M = 32
D = 128
DE = 16
GW = 144
CW = 144
XW = 16

NUM_WORKERS = 32
ROWS_PER_TILE = N_PAD // NUM_WORKERS
TAIL = N_TOTAL - (NUM_WORKERS - 1) * ROWS_PER_TILE
C = 8
HALF = C * M // 2


def _tok_kernel(tok_ref, geo_ref, ln1g_ref, ln1b_ref, wkvT_ref, bkv_ref,
                wattn_ref, g_ref):
    x = tok_ref[...]
    mu = jnp.mean(x, axis=-1, keepdims=True)
    var = jnp.mean((x - mu) ** 2, axis=-1, keepdims=True)
    xn = (x - mu) * lax.rsqrt(var + 1e-5) * ln1g_ref[...] + ln1b_ref[...]
    kv = jnp.dot(xn, wkvT_ref[...], preferred_element_type=jnp.float32)
    kv = kv + bkv_ref[...]
    k = kv[:, :D]
    v = kv[:, D:]
    kw = jnp.sum(k * wattn_ref[...], axis=-1, keepdims=True)
    g_ref[:, 0:D] = v
    g_ref[:, D:D + 1] = kw
    g_ref[:, D + 1:D + 4] = geo_ref[...]
    g_ref[:, D + 4:GW] = jnp.zeros((B1, GW - D - 4), jnp.float32)


def _edge_kernel(e_ref, p_ref, bdwv_ref, bvet_ref, wew_ref, c0_ref,
                 gt_ref, bt_ref, ve_ref, lg_ref):
    e = e_ref[...]
    mexp = jnp.dot(e, p_ref[...], preferred_element_type=jnp.float32)
    vexp = jnp.dot(e * e, p_ref[...], preferred_element_type=jnp.float32)
    vexp = vexp - mexp * mexp
    en = (e - mexp) * lax.rsqrt(vexp + 1e-5) * gt_ref[...] + bt_ref[...]
    ve_ref[...] = jnp.dot(en, bdwv_ref[...],
                          preferred_element_type=jnp.float32) + bvet_ref[...]
    ew = jnp.dot(en, wew_ref[...], preferred_element_type=jnp.float32)
    lg_ref[...] = ew + c0_ref[...]


def _sc_body(idx_hbm, lg_hbm, ve_hbm, g_hbm, out_hbm, aux_hbm,
             idx_all, lg_all, gb0a, gb0b, gb1a, gb1b, vb0, vb1,
             ctx0, ctx1, aux0, aux1, semg0, semg1, semo0, semo1):
    cid = lax.axis_index("c")
    sid = lax.axis_index("s")
    wid = sid * 2 + cid
    base = wid * ROWS_PER_TILE
    valid = jnp.maximum(jnp.minimum(base + ROWS_PER_TILE, N_TOTAL) - base, 0)
    nct = (valid + C - 1) // C

    gbufs = ((gb0a, gb0b), (gb1a, gb1b))
    vbufs = (vb0, vb1)
    ctxs = (ctx0, ctx1)
    auxs = (aux0, aux1)
    semgs = (semg0, semg1)
    semos = (semo0, semo1)

    full = valid == ROWS_PER_TILE

    @pl.when(full)
    def _stage_full():
        pltpu.sync_copy(idx_hbm.at[pl.ds(base * M, ROWS_PER_TILE * M)], idx_all)
        pltpu.sync_copy(lg_hbm.at[pl.ds(base * M, ROWS_PER_TILE * M)], lg_all)

    @pl.when(jnp.logical_not(full))
    def _stage_tail():
        pltpu.sync_copy(idx_hbm.at[pl.ds(base * M, TAIL * M)],
                        idx_all.at[pl.ds(0, TAIL * M)])
        pltpu.sync_copy(lg_hbm.at[pl.ds(base * M, TAIL * M)],
                        lg_all.at[pl.ds(0, TAIL * M)])

    def issue(g, b):
        off = g * C * M
        rb = base + g * C
        pltpu.async_copy(g_hbm.at[idx_all.at[pl.ds(off, HALF)]],
                         gbufs[b][0], semgs[b])
        pltpu.async_copy(g_hbm.at[idx_all.at[pl.ds(off + HALF, HALF)]],
                         gbufs[b][1], semgs[b])
        pltpu.async_copy(ve_hbm.at[pl.ds(rb, C)], vbufs[b], semgs[b])

    def wait_in(g, b):
        off = g * C * M
        rb = base + g * C
        pltpu.make_async_copy(g_hbm.at[idx_all.at[pl.ds(off, HALF)]],
                              gbufs[b][0], semgs[b]).wait()
        pltpu.make_async_copy(g_hbm.at[idx_all.at[pl.ds(off + HALF, HALF)]],
                              gbufs[b][1], semgs[b]).wait()
        pltpu.make_async_copy(ve_hbm.at[pl.ds(rb, C)], vbufs[b], semgs[b]).wait()

    def wait_out(rb, b):
        pltpu.make_async_copy(ctxs[b], out_hbm.at[pl.ds(rb, C)], semos[b]).wait()
        pltpu.make_async_copy(auxs[b], aux_hbm.at[pl.ds(rb, C)], semos[b]).wait()

    def compute(g, b):
        ctx_v = ctxs[b]
        vb = vbufs[b]
        for r in range(C):
            gbuf = gbufs[b][r // (C // 2)]
            row0 = (r % (C // 2)) * M
            lrow = g * C + r
            ids_lo = lax.iota(jnp.int32, 16) + row0
            ids_hi = ids_lo + 16
            colkw = jnp.full((16,), D, jnp.int32)
            kw_lo = plsc.load_gather(gbuf, [ids_lo, colkw])
            kw_hi = plsc.load_gather(gbuf, [ids_hi, colkw])
            t_lo = lg_all[pl.ds(lrow * M, 16)] + kw_lo
            t_hi = lg_all[pl.ds(lrow * M + 16, 16)] + kw_hi
            mx = jnp.maximum(jnp.max(t_lo), jnp.max(t_hi))
            e_lo = jnp.exp(t_lo - mx)
            e_hi = jnp.exp(t_hi - mx)
            sv = jnp.broadcast_to(jnp.sum(e_lo) + jnp.sum(e_hi), (16,))
            a_lo = e_lo / sv
            a_hi = e_hi / sv
            accs = [jnp.zeros((16,), jnp.float32) for _ in range(9)]
            acc_e = jnp.zeros((16,), jnp.float32)
            for mm in range(M):
                a = a_lo[mm] if mm < 16 else a_hi[mm - 16]
                grow = row0 + mm
                for kk in range(9):
                    accs[kk] = accs[kk] + a * gbuf[grow, pl.ds(kk * 16, 16)]
                acc_e = acc_e + a * vb[r, mm, pl.ds(0, 16)]
            for kk in range(8):
                ctx_v[r, pl.ds(kk * 16, 16)] = accs[kk]
            ctx_v[r, pl.ds(D, 16)] = acc_e
            auxs[b][r, pl.ds(0, 16)] = accs[8]

    issue(0, 0)

    def outer(g2, carry):
        for b in (0, 1):
            g = g2 * 2 + b

            @pl.when(g < nct)
            def _process():
                @pl.when(g + 1 < nct)
                def _prefetch():
                    issue(g + 1, 1 - b)

                wait_in(g, b)
                rb = base + g * C

                @pl.when(g >= 2)
                def _drain_prev_out():
                    wait_out(rb, b)

                compute(g, b)
                pltpu.async_copy(ctxs[b], out_hbm.at[pl.ds(rb, C)], semos[b])
                pltpu.async_copy(auxs[b], aux_hbm.at[pl.ds(rb, C)], semos[b])
        return carry

    lax.fori_loop(0, (nct + 1) // 2, outer, 0)
    wait_out(base, 0)
    wait_out(base, 1)


_sc_call_cache = []


def _sc_call(idx_flat, lg_flat, ve3, G):
    if not _sc_call_cache:
        _sc_call_cache.append(pl.kernel(
            _sc_body,
            out_type=(jax.ShapeDtypeStruct((N_TOTAL, CW), jnp.float32),
                      jax.ShapeDtypeStruct((N_TOTAL, XW), jnp.float32)),
            mesh=plsc.VectorSubcoreMesh(core_axis_name="c", subcore_axis_name="s"),
            scratch_types=[
                pltpu.VMEM((ROWS_PER_TILE * M,), jnp.int32),
                pltpu.VMEM((ROWS_PER_TILE * M,), jnp.float32),
                pltpu.VMEM((HALF, GW), jnp.float32),
                pltpu.VMEM((HALF, GW), jnp.float32),
                pltpu.VMEM((HALF, GW), jnp.float32),
                pltpu.VMEM((HALF, GW), jnp.float32),
                pltpu.VMEM((C, M, DE), jnp.float32),
                pltpu.VMEM((C, M, DE), jnp.float32),
                pltpu.VMEM((C, CW), jnp.float32),
                pltpu.VMEM((C, CW), jnp.float32),
                pltpu.VMEM((C, XW), jnp.float32),
                pltpu.VMEM((C, XW), jnp.float32),
                pltpu.SemaphoreType.DMA,
                pltpu.SemaphoreType.DMA,
                pltpu.SemaphoreType.DMA,
                pltpu.SemaphoreType.DMA,
            ],
            compiler_params=pltpu.CompilerParams(
                use_tc_tiling_on_sc=False, needs_layout_passes=False),
        ))
    return _sc_call_cache[0](idx_flat, lg_flat, ve3, G)


def _mlp_kernel(ctx_ref, tok_ref, fc1T_ref, fc1b_ref, lng_ref, lnb_ref,
                fc2T_ref, fc2b_ref, out_ref):
    hin = ctx_ref[...]
    h = jnp.dot(hin, fc1T_ref[...], preferred_element_type=jnp.float32)
    h = h + fc1b_ref[...]
    h = 0.5 * h * (1.0 + lax.erf(h * 0.7071067811865476))
    mu = jnp.mean(h, axis=-1, keepdims=True)
    var = jnp.mean((h - mu) ** 2, axis=-1, keepdims=True)
    h = (h - mu) * lax.rsqrt(var + 1e-5) * lng_ref[...] + lnb_ref[...]
    h = jnp.dot(h, fc2T_ref[...], preferred_element_type=jnp.float32)
    h = h + fc2b_ref[...]
    out_ref[...] = h + tok_ref[...]


def _row_spec(block, width):
    return pl.BlockSpec((block, width), lambda i: (i, 0))


def _full_spec(shape):
    return pl.BlockSpec(shape, lambda i: tuple(0 for _ in shape))


def kernel(token_embs, geo_feats, edge_feats, ln1_g, ln1_b, W_qkv, b_qkv,
           ln_e_g, ln_e_b, W_qkv_e, b_qkv_e, w_attn, w_eattn,
           W_gate_w, W_gate_b, fc1_W, fc1_b, mlp_ln_g, mlp_ln_b, fc2_W, fc2_b,
           neighbor_indices, batch_idx, neighbor_masks):
    f32 = jnp.float32
    N = token_embs.shape[0]

    idx_flat = neighbor_indices.astype(jnp.int32).reshape(N * M)

    WkvT = W_qkv[D:].T
    bkv = b_qkv[D:].reshape(1, 2 * D)
    wattn2 = w_attn.reshape(1, D)

    G = pl.pallas_call(
        _tok_kernel,
        grid=(NB1,),
        in_specs=[
            _row_spec(B1, D),
            _row_spec(B1, 3),
            _full_spec((1, D)),
            _full_spec((1, D)),
            _full_spec((D, 2 * D)),
            _full_spec((1, 2 * D)),
            _full_spec((1, D)),
        ],
        out_specs=_row_spec(B1, GW),
        out_shape=jax.ShapeDtypeStruct((N, GW), f32),
    )(token_embs, geo_feats, ln1_g.reshape(1, D), ln1_b.reshape(1, D),
      WkvT, bkv, wattn2)

    ER = 8
    RW = ER * DE
    NR = N * M // ER
    B2 = 2000
    NB2 = NR // B2
    eye8 = jnp.eye(ER, dtype=f32)
    P = jnp.kron(eye8, jnp.full((DE, DE), 1.0 / DE, f32))
    WqeT = W_qkv_e[:DE].T
    WveT = W_qkv_e[DE:].T
    BDWv = jnp.kron(eye8, WveT)
    bvet = jnp.tile(b_qkv_e[DE:], ER).reshape(1, RW)
    wcomb = WqeT @ w_eattn
    Wew = jnp.kron(eye8, wcomb.reshape(DE, 1))
    c0 = jnp.full((1, ER), jnp.dot(b_qkv_e[:DE], w_eattn), f32)
    gt = jnp.tile(ln_e_g, ER).reshape(1, RW)
    bt = jnp.tile(ln_e_b, ER).reshape(1, RW)
    e8 = edge_feats.reshape(NR, RW)

    ve2, lg2 = pl.pallas_call(
        _edge_kernel,
        grid=(NB2,),
        in_specs=[
            _row_spec(B2, RW),
            _full_spec((RW, RW)),
            _full_spec((RW, RW)),
            _full_spec((1, RW)),
            _full_spec((RW, ER)),
            _full_spec((1, ER)),
            _full_spec((1, RW)),
            _full_spec((1, RW)),
        ],
        out_specs=[_row_spec(B2, RW), _row_spec(B2, ER)],
        out_shape=[
            jax.ShapeDtypeStruct((NR, RW), f32),
            jax.ShapeDtypeStruct((NR, ER), f32),
        ],
    )(e8, P, BDWv, bvet, Wew, c0, gt, bt)

    ve3 = ve2.reshape(N, M, DE)
    lg_flat = lg2.reshape(N * M)

    ctx, aux = _sc_call(idx_flat, lg_flat, ve3, G)

    out = pl.pallas_call(
        _mlp_kernel,
        grid=(NB1,),
        in_specs=[
            _row_spec(B1, CW),
            _row_spec(B1, D),
            _full_spec((D + DE, D)),
            _full_spec((1, D)),
            _full_spec((1, D)),
            _full_spec((1, D)),
            _full_spec((D, D)),
            _full_spec((1, D)),
        ],
        out_specs=_row_spec(B1, D),
        out_shape=jax.ShapeDtypeStruct((N, D), f32),
    )(ctx, token_embs, fc1_W.T, fc1_b.reshape(1, D), mlp_ln_g.reshape(1, D),
      mlp_ln_b.reshape(1, D), fc2_W.T, fc2_b.reshape(1, D))

    geo_context = aux[:, 1:4]
    return out, geo_context


# speedup vs baseline: 1.2685x; 1.2685x over previous
import jax
import jax.numpy as jnp
from jax import lax
from jax.experimental import pallas as pl
from jax.experimental.pallas import tpu as pltpu
from jax.experimental.pallas import tpu_sc as plsc

N_TOTAL = 10000
N_PAD = 10240
B1 = 400
NB1 = N_TOTAL // B1
M = 32
D = 128
DE = 16
GW = 144
CW = 144
XW = 16

NUM_WORKERS = 32
ROWS_PER_TILE = N_PAD // NUM_WORKERS
TAIL = N_TOTAL - (NUM_WORKERS - 1) * ROWS_PER_TILE
C = 8
HALF = C * M // 2


def _tok_kernel(tok_ref, geo_ref, ln1g_ref, ln1b_ref, wkvT_ref, bkv_ref,
                wattn_ref, g_ref):
    x = tok_ref[...]
    mu = jnp.mean(x, axis=-1, keepdims=True)
    var = jnp.mean((x - mu) ** 2, axis=-1, keepdims=True)
    xn = (x - mu) * lax.rsqrt(var + 1e-5) * ln1g_ref[...] + ln1b_ref[...]
    kv = jnp.dot(xn, wkvT_ref[...], preferred_element_type=jnp.float32)
    kv = kv + bkv_ref[...]
    k = kv[:, :D]
    v = kv[:, D:]
    kw = jnp.sum(k * wattn_ref[...], axis=-1, keepdims=True)
    g_ref[:, 0:D] = v
    g_ref[:, D:D + 1] = kw
    g_ref[:, D + 1:D + 4] = geo_ref[...]
    g_ref[:, D + 4:GW] = jnp.zeros((B1, GW - D - 4), jnp.float32)


MPG = 4


def _edge_kernel(e_ref, wv_ref, bve_ref, wcomb_ref, c0_ref,
                 g_ref, b_ref, ve_ref, lg_ref):
    i = pl.program_id(0)
    for t in range(MPG):
        e = e_ref[pl.ds(t * DE, DE), :]
        mu = jnp.mean(e, axis=0, keepdims=True)
        var = jnp.mean((e - mu) ** 2, axis=0, keepdims=True)
        en = (e - mu) * lax.rsqrt(var + 1e-5) * g_ref[...] + b_ref[...]
        ve_ref[pl.ds(t * DE, DE), :] = jnp.dot(
            wv_ref[...], en, preferred_element_type=jnp.float32) + bve_ref[...]
        lg = jnp.dot(wcomb_ref[...], en,
                     preferred_element_type=jnp.float32) + c0_ref[...]
        lg_ref[pl.ds(i * MPG + t, 1), :] = lg


def _sc_body(idx_hbm, lg_hbm, ve_hbm, g_hbm, out_hbm, aux_hbm,
             idx_all, lgbuf, gb0a, gb0b, gb1a, gb1b, vb0, vb1,
             ctx0, ctx1, aux0, aux1, semg0, semg1, semo0, semo1):
    cid = lax.axis_index("c")
    sid = lax.axis_index("s")
    wid = sid * 2 + cid
    base = wid * ROWS_PER_TILE
    valid = jnp.maximum(jnp.minimum(base + ROWS_PER_TILE, N_TOTAL) - base, 0)
    nct = (valid + C - 1) // C

    gbufs = ((gb0a, gb0b), (gb1a, gb1b))
    vbs = (vb0, vb1)
    ctxs = (ctx0, ctx1)
    auxs = (aux0, aux1)
    semgs = (semg0, semg1)
    semos = (semo0, semo1)

    full = valid == ROWS_PER_TILE

    @pl.when(full)
    def _stage_full():
        pltpu.sync_copy(idx_hbm.at[pl.ds(base * M, ROWS_PER_TILE * M)], idx_all)
        pltpu.sync_copy(lg_hbm.at[:, pl.ds(base, ROWS_PER_TILE)], lgbuf)

    @pl.when(jnp.logical_not(full))
    def _stage_tail():
        pltpu.sync_copy(idx_hbm.at[pl.ds(base * M, TAIL * M)],
                        idx_all.at[pl.ds(0, TAIL * M)])
        pltpu.sync_copy(lg_hbm.at[:, pl.ds(base, TAIL)],
                        lgbuf.at[:, pl.ds(0, TAIL)])

    def issue_gather(g, b):
        off = g * C * M
        rb = base + g * C
        pltpu.async_copy(g_hbm.at[idx_all.at[pl.ds(off, HALF)]],
                         gbufs[b][0], semgs[b])
        pltpu.async_copy(g_hbm.at[idx_all.at[pl.ds(off + HALF, HALF)]],
                         gbufs[b][1], semgs[b])
        pltpu.async_copy(ve_hbm.at[:, pl.ds(rb, C)], vbs[b], semgs[b])

    def wait_gather(g, b):
        off = g * C * M
        rb = base + g * C
        pltpu.make_async_copy(g_hbm.at[idx_all.at[pl.ds(off, HALF)]],
                              gbufs[b][0], semgs[b]).wait()
        pltpu.make_async_copy(g_hbm.at[idx_all.at[pl.ds(off + HALF, HALF)]],
                              gbufs[b][1], semgs[b]).wait()
        pltpu.make_async_copy(ve_hbm.at[:, pl.ds(rb, C)], vbs[b], semgs[b]).wait()

    def wait_out(b):
        pltpu.make_async_copy(ctxs[b], out_hbm.at[pl.ds(base, C)], semos[b]).wait()
        pltpu.make_async_copy(auxs[b], aux_hbm.at[pl.ds(base, C)], semos[b]).wait()

    def compute(g, b):
        vb = vbs[b]
        ctx_v = ctxs[b]
        i16 = lax.iota(jnp.int32, 16)
        for r in range(C):
            gbuf = gbufs[b][r // (C // 2)]
            row0 = (r % (C // 2)) * M
            lrow = g * C + r
            rc = jnp.full((16,), r, jnp.int32)
            colkw = jnp.full((16,), D, jnp.int32)
            lcol = jnp.full((16,), lrow, jnp.int32)
            kw_lo = plsc.load_gather(gbuf, [i16 + row0, colkw])
            kw_hi = plsc.load_gather(gbuf, [i16 + row0 + 16, colkw])
            t_lo = plsc.load_gather(lgbuf, [i16, lcol]) + kw_lo
            t_hi = plsc.load_gather(lgbuf, [i16 + 16, lcol]) + kw_hi
            mx = jnp.maximum(jnp.max(t_lo), jnp.max(t_hi))
            e_lo = jnp.exp(t_lo - mx)
            e_hi = jnp.exp(t_hi - mx)
            sv = jnp.broadcast_to(jnp.sum(e_lo) + jnp.sum(e_hi), (16,))
            a_lo = e_lo / sv
            a_hi = e_hi / sv
            accs = [jnp.zeros((16,), jnp.float32) for _ in range(9)]
            acc_e = jnp.zeros((16,), jnp.float32)
            for mm in range(M):
                a = a_lo[mm] if mm < 16 else a_hi[mm - 16]
                grow = row0 + mm
                for kk in range(9):
                    accs[kk] = accs[kk] + a * gbuf[grow, pl.ds(kk * 16, 16)]
                acc_e = acc_e + a * plsc.load_gather(vb, [i16 + mm * DE, rc])
            for kk in range(8):
                ctx_v[r, pl.ds(kk * 16, 16)] = accs[kk]
            ctx_v[r, pl.ds(D, 16)] = acc_e
            auxs[b][r, pl.ds(0, 16)] = accs[8]

    issue_gather(0, 0)

    def outer(g2, carry):
        for b in (0, 1):
            g = g2 * 2 + b

            @pl.when(g < nct)
            def _process():
                @pl.when(g + 1 < nct)
                def _prefetch():
                    issue_gather(g + 1, 1 - b)

                wait_gather(g, b)
                rb = base + g * C

                @pl.when(g >= 2)
                def _drain_prev_out():
                    wait_out(b)

                compute(g, b)
                pltpu.async_copy(ctxs[b], out_hbm.at[pl.ds(rb, C)], semos[b])
                pltpu.async_copy(auxs[b], aux_hbm.at[pl.ds(rb, C)], semos[b])
        return carry

    lax.fori_loop(0, (nct + 1) // 2, outer, 0)
    wait_out(0)
    wait_out(1)


_sc_call_cache = []


def _sc_call(idx_flat, lg_t, ve_t, G):
    if not _sc_call_cache:
        _sc_call_cache.append(pl.kernel(
            _sc_body,
            out_type=(jax.ShapeDtypeStruct((N_TOTAL, CW), jnp.float32),
                      jax.ShapeDtypeStruct((N_TOTAL, XW), jnp.float32)),
            mesh=plsc.VectorSubcoreMesh(core_axis_name="c", subcore_axis_name="s"),
            scratch_types=[
                pltpu.VMEM((ROWS_PER_TILE * M,), jnp.int32),
                pltpu.VMEM((M, ROWS_PER_TILE), jnp.float32),
                pltpu.VMEM((HALF, GW), jnp.float32),
                pltpu.VMEM((HALF, GW), jnp.float32),
                pltpu.VMEM((HALF, GW), jnp.float32),
                pltpu.VMEM((HALF, GW), jnp.float32),
                pltpu.VMEM((M * DE, C), jnp.float32),
                pltpu.VMEM((M * DE, C), jnp.float32),
                pltpu.VMEM((C, CW), jnp.float32),
                pltpu.VMEM((C, CW), jnp.float32),
                pltpu.VMEM((C, XW), jnp.float32),
                pltpu.VMEM((C, XW), jnp.float32),
                pltpu.SemaphoreType.DMA,
                pltpu.SemaphoreType.DMA,
                pltpu.SemaphoreType.DMA,
                pltpu.SemaphoreType.DMA,
            ],
            compiler_params=pltpu.CompilerParams(
                use_tc_tiling_on_sc=False, needs_layout_passes=False),
        ))
    return _sc_call_cache[0](idx_flat, lg_t, ve_t, G)


def _mlp_kernel(ctx_ref, tok_ref, fc1T_ref, fc1b_ref, lng_ref, lnb_ref,
                fc2T_ref, fc2b_ref, out_ref):
    hin = ctx_ref[...]
    h = jnp.dot(hin, fc1T_ref[...], preferred_element_type=jnp.float32)
    h = h + fc1b_ref[...]
    h = 0.5 * h * (1.0 + lax.erf(h * 0.7071067811865476))
    mu = jnp.mean(h, axis=-1, keepdims=True)
    var = jnp.mean((h - mu) ** 2, axis=-1, keepdims=True)
    h = (h - mu) * lax.rsqrt(var + 1e-5) * lng_ref[...] + lnb_ref[...]
    h = jnp.dot(h, fc2T_ref[...], preferred_element_type=jnp.float32)
    h = h + fc2b_ref[...]
    out_ref[...] = h + tok_ref[...]


def _row_spec(block, width):
    return pl.BlockSpec((block, width), lambda i: (i, 0))


def _full_spec(shape):
    return pl.BlockSpec(shape, lambda i: tuple(0 for _ in shape))


def kernel(token_embs, geo_feats, edge_feats, ln1_g, ln1_b, W_qkv, b_qkv,
           ln_e_g, ln_e_b, W_qkv_e, b_qkv_e, w_attn, w_eattn,
           W_gate_w, W_gate_b, fc1_W, fc1_b, mlp_ln_g, mlp_ln_b, fc2_W, fc2_b,
           neighbor_indices, batch_idx, neighbor_masks):
    f32 = jnp.float32
    N = token_embs.shape[0]

    idx_flat = neighbor_indices.astype(jnp.int32).reshape(N * M)

    WkvT = W_qkv[D:].T
    bkv = b_qkv[D:].reshape(1, 2 * D)
    wattn2 = w_attn.reshape(1, D)

    G = pl.pallas_call(
        _tok_kernel,
        grid=(NB1,),
        in_specs=[
            _row_spec(B1, D),
            _row_spec(B1, 3),
            _full_spec((1, D)),
            _full_spec((1, D)),
            _full_spec((D, 2 * D)),
            _full_spec((1, 2 * D)),
            _full_spec((1, D)),
        ],
        out_specs=_row_spec(B1, GW),
        out_shape=jax.ShapeDtypeStruct((N, GW), f32),
    )(token_embs, geo_feats, ln1_g.reshape(1, D), ln1_b.reshape(1, D),
      WkvT, bkv, wattn2)

    et = edge_feats.transpose(1, 2, 0).reshape(M * DE, N)
    Wv = W_qkv_e[DE:]
    bve = b_qkv_e[DE:].reshape(DE, 1)
    wcomb = (w_eattn @ W_qkv_e[:DE]).reshape(1, DE)
    c0 = jnp.dot(b_qkv_e[:DE], w_eattn).reshape(1, 1)
    gcol = ln_e_g.reshape(DE, 1)
    bcol = ln_e_b.reshape(DE, 1)

    BR = MPG * DE
    ve_t, lg_t = pl.pallas_call(
        _edge_kernel,
        grid=(M // MPG,),
        in_specs=[
            pl.BlockSpec((BR, N), lambda i: (i, 0)),
            _full_spec((DE, DE)),
            _full_spec((DE, 1)),
            _full_spec((1, DE)),
            _full_spec((1, 1)),
            _full_spec((DE, 1)),
            _full_spec((DE, 1)),
        ],
        out_specs=[pl.BlockSpec((BR, N), lambda i: (i, 0)),
                   pl.BlockSpec((M, N), lambda i: (0, 0))],
        out_shape=[
            jax.ShapeDtypeStruct((M * DE, N), f32),
            jax.ShapeDtypeStruct((M, N), f32),
        ],
    )(et, Wv, bve, wcomb, c0, gcol, bcol)

    ctx, aux = _sc_call(idx_flat, lg_t, ve_t, G)

    out = pl.pallas_call(
        _mlp_kernel,
        grid=(NB1,),
        in_specs=[
            _row_spec(B1, CW),
            _row_spec(B1, D),
            _full_spec((D + DE, D)),
            _full_spec((1, D)),
            _full_spec((1, D)),
            _full_spec((1, D)),
            _full_spec((D, D)),
            _full_spec((1, D)),
        ],
        out_specs=_row_spec(B1, D),
        out_shape=jax.ShapeDtypeStruct((N, D), f32),
    )(ctx, token_embs, fc1_W.T, fc1_b.reshape(1, D), mlp_ln_g.reshape(1, D),
      mlp_ln_b.reshape(1, D), fc2_W.T, fc2_b.reshape(1, D))

    geo_context = aux[:, 1:4]
    return out, geo_context


# speedup vs baseline: 1.3246x; 1.0442x over previous
import jax
import jax.numpy as jnp
import numpy as np
from jax import lax
from jax.experimental import pallas as pl
from jax.experimental.pallas import tpu as pltpu
from jax.experimental.pallas import tpu_sc as plsc

N_TOTAL = 10000
N_PAD = 10240
B1 = 400
NB1 = N_TOTAL // B1
M = 32
D = 128
DE = 16
GW = 144
SW = 16
CW = 144
XW = 16

_PERM_SRC = np.empty(D, np.int32)
for _k in range(4):
    for _i in range(16):
        _PERM_SRC[32 * _k + 2 * _i] = 32 * _k + _i
        _PERM_SRC[32 * _k + 2 * _i + 1] = 32 * _k + 16 + _i
_PERM_MAT = np.zeros((D, D), np.float32)
_PERM_MAT[_PERM_SRC, np.arange(D)] = 1.0
INV_PERM = np.empty(D, np.int32)
INV_PERM[_PERM_SRC] = np.arange(D)

NUM_WORKERS = 32
ROWS_PER_TILE = N_PAD // NUM_WORKERS
TAIL = N_TOTAL - (NUM_WORKERS - 1) * ROWS_PER_TILE
C = 8
HALF = C * M // 2


def _tok_kernel(tok_ref, geo_ref, ln1g_ref, ln1b_ref, wkvT_ref, bkv_ref,
                wattn_ref, pm_ref, gv_ref, gs_ref):
    x = tok_ref[...]
    mu = jnp.mean(x, axis=-1, keepdims=True)
    var = jnp.mean((x - mu) ** 2, axis=-1, keepdims=True)
    xn = (x - mu) * lax.rsqrt(var + 1e-5) * ln1g_ref[...] + ln1b_ref[...]
    kv = jnp.dot(xn, wkvT_ref[...], preferred_element_type=jnp.float32)
    kv = kv + bkv_ref[...]
    k = kv[:, :D]
    v = kv[:, D:]
    kw = jnp.sum(k * wattn_ref[...], axis=-1, keepdims=True)
    vperm = jnp.dot(v, pm_ref[...], preferred_element_type=jnp.float32)
    gv_ref[...] = vperm.astype(jnp.bfloat16)
    gs_ref[:, 0:1] = kw
    gs_ref[:, 1:4] = geo_ref[...]
    gs_ref[:, 4:SW] = jnp.zeros((B1, SW - 4), jnp.float32)


MPG = 4


def _edge_kernel(e_ref, wv_ref, bve_ref, wcomb_ref, c0_ref,
                 g_ref, b_ref, ve_ref, lg_ref):
    i = pl.program_id(0)
    for t in range(MPG):
        e = e_ref[pl.ds(t * DE, DE), :]
        mu = jnp.mean(e, axis=0, keepdims=True)
        var = jnp.mean((e - mu) ** 2, axis=0, keepdims=True)
        en = (e - mu) * lax.rsqrt(var + 1e-5) * g_ref[...] + b_ref[...]
        ve_ref[pl.ds(t * DE, DE), :] = jnp.dot(
            wv_ref[...], en, preferred_element_type=jnp.float32) + bve_ref[...]
        lg = jnp.dot(wcomb_ref[...], en,
                     preferred_element_type=jnp.float32) + c0_ref[...]
        lg_ref[pl.ds(i * MPG + t, 1), :] = lg


def _sc_body(idx_hbm, lg_hbm, ve_hbm, gv_hbm, gs_hbm, out_hbm, aux_hbm,
             idx_all, lgbuf, gv0a, gv0b, gv1a, gv1b, gs0a, gs0b, gs1a, gs1b,
             vb0, vb1, ctx0, ctx1, aux0, aux1, semg0, semg1, semo0, semo1):
    cid = lax.axis_index("c")
    sid = lax.axis_index("s")
    wid = sid * 2 + cid
    base = wid * ROWS_PER_TILE
    valid = jnp.maximum(jnp.minimum(base + ROWS_PER_TILE, N_TOTAL) - base, 0)
    nct = (valid + C - 1) // C

    gvs = ((gv0a, gv0b), (gv1a, gv1b))
    gss = ((gs0a, gs0b), (gs1a, gs1b))
    vbs = (vb0, vb1)
    ctxs = (ctx0, ctx1)
    auxs = (aux0, aux1)
    semgs = (semg0, semg1)
    semos = (semo0, semo1)

    full = valid == ROWS_PER_TILE

    @pl.when(full)
    def _stage_full():
        pltpu.sync_copy(idx_hbm.at[pl.ds(base * M, ROWS_PER_TILE * M)], idx_all)
        pltpu.sync_copy(lg_hbm.at[:, pl.ds(base, ROWS_PER_TILE)], lgbuf)

    @pl.when(jnp.logical_not(full))
    def _stage_tail():
        pltpu.sync_copy(idx_hbm.at[pl.ds(base * M, TAIL * M)],
                        idx_all.at[pl.ds(0, TAIL * M)])
        pltpu.sync_copy(lg_hbm.at[:, pl.ds(base, TAIL)],
                        lgbuf.at[:, pl.ds(0, TAIL)])

    def issue_gather(g, b):
        off = g * C * M
        rb = base + g * C
        lo = idx_all.at[pl.ds(off, HALF)]
        hi = idx_all.at[pl.ds(off + HALF, HALF)]
        pltpu.async_copy(gv_hbm.at[lo], gvs[b][0], semgs[b])
        pltpu.async_copy(gv_hbm.at[hi], gvs[b][1], semgs[b])
        pltpu.async_copy(gs_hbm.at[lo], gss[b][0], semgs[b])
        pltpu.async_copy(gs_hbm.at[hi], gss[b][1], semgs[b])
        pltpu.async_copy(ve_hbm.at[:, pl.ds(rb, C)], vbs[b], semgs[b])

    def wait_gather(g, b):
        off = g * C * M
        rb = base + g * C
        lo = idx_all.at[pl.ds(off, HALF)]
        hi = idx_all.at[pl.ds(off + HALF, HALF)]
        pltpu.make_async_copy(gv_hbm.at[lo], gvs[b][0], semgs[b]).wait()
        pltpu.make_async_copy(gv_hbm.at[hi], gvs[b][1], semgs[b]).wait()
        pltpu.make_async_copy(gs_hbm.at[lo], gss[b][0], semgs[b]).wait()
        pltpu.make_async_copy(gs_hbm.at[hi], gss[b][1], semgs[b]).wait()
        pltpu.make_async_copy(ve_hbm.at[:, pl.ds(rb, C)], vbs[b], semgs[b]).wait()

    def wait_out(b):
        pltpu.make_async_copy(ctxs[b], out_hbm.at[pl.ds(base, C)], semos[b]).wait()
        pltpu.make_async_copy(auxs[b], aux_hbm.at[pl.ds(base, C)], semos[b]).wait()

    def compute(g, b):
        vb = vbs[b]
        ctx_v = ctxs[b]
        i16 = lax.iota(jnp.int32, 16)
        for r in range(C):
            half = r // (C // 2)
            gv = gvs[b][half]
            gs = gss[b][half]
            row0 = (r % (C // 2)) * M
            lrow = g * C + r
            rc = jnp.full((16,), r, jnp.int32)
            col0 = jnp.full((16,), 0, jnp.int32)
            lcol = jnp.full((16,), lrow, jnp.int32)
            kw_lo = plsc.load_gather(gs, [i16 + row0, col0])
            kw_hi = plsc.load_gather(gs, [i16 + row0 + 16, col0])
            t_lo = plsc.load_gather(lgbuf, [i16, lcol]) + kw_lo
            t_hi = plsc.load_gather(lgbuf, [i16 + 16, lcol]) + kw_hi
            mx = jnp.maximum(jnp.max(t_lo), jnp.max(t_hi))
            e_lo = jnp.exp(t_lo - mx)
            e_hi = jnp.exp(t_hi - mx)
            sv = jnp.broadcast_to(jnp.sum(e_lo) + jnp.sum(e_hi), (16,))
            a_lo = e_lo / sv
            a_hi = e_hi / sv
            accs = [jnp.zeros((16,), jnp.float32) for _ in range(8)]
            acc_s = jnp.zeros((16,), jnp.float32)
            acc_e = jnp.zeros((16,), jnp.float32)
            for mm in range(M):
                a = a_lo[mm] if mm < 16 else a_hi[mm - 16]
                grow = row0 + mm
                acc_s = acc_s + a * gs[grow, pl.ds(0, 16)]
                for kk in range(4):
                    vb32 = gv[grow, pl.ds(kk * 32, 32)]
                    vlo, vhi = plsc.unpack(vb32, format=plsc.PackFormat.INTERLEAVED)
                    accs[2 * kk] = accs[2 * kk] + a * vlo
                    accs[2 * kk + 1] = accs[2 * kk + 1] + a * vhi
                acc_e = acc_e + a * plsc.load_gather(vb, [i16 + mm * DE, rc])
            for kk in range(8):
                ctx_v[r, pl.ds(kk * 16, 16)] = accs[kk]
            ctx_v[r, pl.ds(D, 16)] = acc_e
            auxs[b][r, pl.ds(0, 16)] = acc_s

    issue_gather(0, 0)

    def outer(g2, carry):
        for b in (0, 1):
            g = g2 * 2 + b

            @pl.when(g < nct)
            def _process():
                @pl.when(g + 1 < nct)
                def _prefetch():
                    issue_gather(g + 1, 1 - b)

                wait_gather(g, b)
                rb = base + g * C

                @pl.when(g >= 2)
                def _drain_prev_out():
                    wait_out(b)

                compute(g, b)
                pltpu.async_copy(ctxs[b], out_hbm.at[pl.ds(rb, C)], semos[b])
                pltpu.async_copy(auxs[b], aux_hbm.at[pl.ds(rb, C)], semos[b])
        return carry

    lax.fori_loop(0, (nct + 1) // 2, outer, 0)
    wait_out(0)
    wait_out(1)


_sc_call_cache = []


def _sc_call(idx_flat, lg_t, ve_t, Gv, Gs):
    if not _sc_call_cache:
        _sc_call_cache.append(pl.kernel(
            _sc_body,
            out_type=(jax.ShapeDtypeStruct((N_TOTAL, CW), jnp.float32),
                      jax.ShapeDtypeStruct((N_TOTAL, XW), jnp.float32)),
            mesh=plsc.VectorSubcoreMesh(core_axis_name="c", subcore_axis_name="s"),
            scratch_types=[
                pltpu.VMEM((ROWS_PER_TILE * M,), jnp.int32),
                pltpu.VMEM((M, ROWS_PER_TILE), jnp.float32),
                pltpu.VMEM((HALF, D), jnp.bfloat16),
                pltpu.VMEM((HALF, D), jnp.bfloat16),
                pltpu.VMEM((HALF, D), jnp.bfloat16),
                pltpu.VMEM((HALF, D), jnp.bfloat16),
                pltpu.VMEM((HALF, SW), jnp.float32),
                pltpu.VMEM((HALF, SW), jnp.float32),
                pltpu.VMEM((HALF, SW), jnp.float32),
                pltpu.VMEM((HALF, SW), jnp.float32),
                pltpu.VMEM((M * DE, C), jnp.float32),
                pltpu.VMEM((M * DE, C), jnp.float32),
                pltpu.VMEM((C, CW), jnp.float32),
                pltpu.VMEM((C, CW), jnp.float32),
                pltpu.VMEM((C, XW), jnp.float32),
                pltpu.VMEM((C, XW), jnp.float32),
                pltpu.SemaphoreType.DMA,
                pltpu.SemaphoreType.DMA,
                pltpu.SemaphoreType.DMA,
                pltpu.SemaphoreType.DMA,
            ],
            compiler_params=pltpu.CompilerParams(
                use_tc_tiling_on_sc=False, needs_layout_passes=False),
        ))
    return _sc_call_cache[0](idx_flat, lg_t, ve_t, Gv, Gs)


def _mlp_kernel(ctx_ref, tok_ref, fc1T_ref, fc1b_ref, lng_ref, lnb_ref,
                fc2T_ref, fc2b_ref, out_ref):
    hin = ctx_ref[...]
    h = jnp.dot(hin, fc1T_ref[...], preferred_element_type=jnp.float32)
    h = h + fc1b_ref[...]
    h = 0.5 * h * (1.0 + lax.erf(h * 0.7071067811865476))
    mu = jnp.mean(h, axis=-1, keepdims=True)
    var = jnp.mean((h - mu) ** 2, axis=-1, keepdims=True)
    h = (h - mu) * lax.rsqrt(var + 1e-5) * lng_ref[...] + lnb_ref[...]
    h = jnp.dot(h, fc2T_ref[...], preferred_element_type=jnp.float32)
    h = h + fc2b_ref[...]
    out_ref[...] = h + tok_ref[...]


def _row_spec(block, width):
    return pl.BlockSpec((block, width), lambda i: (i, 0))


def _full_spec(shape):
    return pl.BlockSpec(shape, lambda i: tuple(0 for _ in shape))


def kernel(token_embs, geo_feats, edge_feats, ln1_g, ln1_b, W_qkv, b_qkv,
           ln_e_g, ln_e_b, W_qkv_e, b_qkv_e, w_attn, w_eattn,
           W_gate_w, W_gate_b, fc1_W, fc1_b, mlp_ln_g, mlp_ln_b, fc2_W, fc2_b,
           neighbor_indices, batch_idx, neighbor_masks):
    f32 = jnp.float32
    N = token_embs.shape[0]

    idx_flat = neighbor_indices.astype(jnp.int32).reshape(N * M)

    WkvT = W_qkv[D:].T
    bkv = b_qkv[D:].reshape(1, 2 * D)
    wattn2 = w_attn.reshape(1, D)

    Gv, Gs = pl.pallas_call(
        _tok_kernel,
        grid=(NB1,),
        in_specs=[
            _row_spec(B1, D),
            _row_spec(B1, 3),
            _full_spec((1, D)),
            _full_spec((1, D)),
            _full_spec((D, 2 * D)),
            _full_spec((1, 2 * D)),
            _full_spec((1, D)),
            _full_spec((D, D)),
        ],
        out_specs=[_row_spec(B1, D), _row_spec(B1, SW)],
        out_shape=[jax.ShapeDtypeStruct((N, D), jnp.bfloat16),
                   jax.ShapeDtypeStruct((N, SW), f32)],
    )(token_embs, geo_feats, ln1_g.reshape(1, D), ln1_b.reshape(1, D),
      WkvT, bkv, wattn2, jnp.asarray(_PERM_MAT))

    et = edge_feats.transpose(1, 2, 0).reshape(M * DE, N)
    Wv = W_qkv_e[DE:]
    bve = b_qkv_e[DE:].reshape(DE, 1)
    wcomb = (w_eattn @ W_qkv_e[:DE]).reshape(1, DE)
    c0 = jnp.dot(b_qkv_e[:DE], w_eattn).reshape(1, 1)
    gcol = ln_e_g.reshape(DE, 1)
    bcol = ln_e_b.reshape(DE, 1)

    BR = MPG * DE
    ve_t, lg_t = pl.pallas_call(
        _edge_kernel,
        grid=(M // MPG,),
        in_specs=[
            pl.BlockSpec((BR, N), lambda i: (i, 0)),
            _full_spec((DE, DE)),
            _full_spec((DE, 1)),
            _full_spec((1, DE)),
            _full_spec((1, 1)),
            _full_spec((DE, 1)),
            _full_spec((DE, 1)),
        ],
        out_specs=[pl.BlockSpec((BR, N), lambda i: (i, 0)),
                   pl.BlockSpec((M, N), lambda i: (0, 0))],
        out_shape=[
            jax.ShapeDtypeStruct((M * DE, N), f32),
            jax.ShapeDtypeStruct((M, N), f32),
        ],
    )(et, Wv, bve, wcomb, c0, gcol, bcol)

    ctx, aux = _sc_call(idx_flat, lg_t, ve_t, Gv, Gs)

    out = pl.pallas_call(
        _mlp_kernel,
        grid=(NB1,),
        in_specs=[
            _row_spec(B1, CW),
            _row_spec(B1, D),
            _full_spec((D + DE, D)),
            _full_spec((1, D)),
            _full_spec((1, D)),
            _full_spec((1, D)),
            _full_spec((D, D)),
            _full_spec((1, D)),
        ],
        out_specs=_row_spec(B1, D),
        out_shape=jax.ShapeDtypeStruct((N, D), f32),
    )(ctx, token_embs, fc1_W.T, fc1_b.reshape(1, D), mlp_ln_g.reshape(1, D),
      mlp_ln_b.reshape(1, D), fc2_W.T, fc2_b.reshape(1, D))

    geo_context = aux[:, 1:4]
    return out, geo_context


# speedup vs baseline: 1.7732x; 1.3387x over previous
import jax
import jax.numpy as jnp
import numpy as np
from jax import lax
from jax.experimental import pallas as pl
from jax.experimental.pallas import tpu as pltpu
from jax.experimental.pallas import tpu_sc as plsc

N_TOTAL = 10000
N_PAD = 10240
B1 = 400
NB1 = N_TOTAL // B1
M = 32
D = 128
DE = 16
GW = 144
SW = 16
CW = 144
XW = 16

_PERM_SRC = np.empty(D, np.int32)
for _k in range(4):
    for _i in range(16):
        _PERM_SRC[32 * _k + 2 * _i] = 32 * _k + _i
        _PERM_SRC[32 * _k + 2 * _i + 1] = 32 * _k + 16 + _i
_PERM_MAT = np.zeros((D, D), np.float32)
_PERM_MAT[_PERM_SRC, np.arange(D)] = 1.0
INV_PERM = np.empty(D, np.int32)
INV_PERM[_PERM_SRC] = np.arange(D)

NUM_WORKERS = 32
ROWS_PER_TILE = N_PAD // NUM_WORKERS
TAIL = N_TOTAL - (NUM_WORKERS - 1) * ROWS_PER_TILE
C = 8
HALF = C * M // 2


def _tok_kernel(tok_ref, geo_ref, ln1g_ref, ln1b_ref, wkvT_ref, bkv_ref,
                wattn_ref, pm_ref, gv_ref, gs_ref):
    x = tok_ref[...]
    mu = jnp.mean(x, axis=-1, keepdims=True)
    var = jnp.mean((x - mu) ** 2, axis=-1, keepdims=True)
    xn = (x - mu) * lax.rsqrt(var + 1e-5) * ln1g_ref[...] + ln1b_ref[...]
    kv = jnp.dot(xn, wkvT_ref[...], preferred_element_type=jnp.float32)
    kv = kv + bkv_ref[...]
    k = kv[:, :D]
    v = kv[:, D:]
    kw = jnp.sum(k * wattn_ref[...], axis=-1, keepdims=True)
    vperm = jnp.dot(v, pm_ref[...], preferred_element_type=jnp.float32)
    gv_ref[...] = vperm.astype(jnp.bfloat16)
    gs_ref[:, 0:1] = kw
    gs_ref[:, 1:4] = geo_ref[...]
    gs_ref[:, 4:SW] = jnp.zeros((B1, SW - 4), jnp.float32)


MPG = 4


def _edge_kernel(e_ref, wv_ref, bve_ref, wcomb_ref, c0_ref,
                 g_ref, b_ref, ve_ref, lg_ref):
    i = pl.program_id(0)
    for t in range(MPG):
        e = e_ref[pl.ds(t * DE, DE), :]
        mu = jnp.mean(e, axis=0, keepdims=True)
        var = jnp.mean((e - mu) ** 2, axis=0, keepdims=True)
        en = (e - mu) * lax.rsqrt(var + 1e-5) * g_ref[...] + b_ref[...]
        ve_ref[pl.ds(t * DE, DE), :] = jnp.dot(
            wv_ref[...], en, preferred_element_type=jnp.float32) + bve_ref[...]
        lg = jnp.dot(wcomb_ref[...], en,
                     preferred_element_type=jnp.float32) + c0_ref[...]
        lg_ref[pl.ds(i * MPG + t, 1), :] = lg


def _sc_body(idx_hbm, lg_hbm, ve_hbm, gv_hbm, gs_hbm, out_hbm, aux_hbm,
             idx_all, lgbuf, gv0a, gv0b, gv1a, gv1b, gs0a, gs0b, gs1a, gs1b,
             vb0, vb1, ctx0, ctx1, aux0, aux1,
             semg0, semg1, semo0, semo1, semv0, semv1):
    cid = lax.axis_index("c")
    sid = lax.axis_index("s")
    wid = sid * 2 + cid
    base = wid * ROWS_PER_TILE
    valid = jnp.maximum(jnp.minimum(base + ROWS_PER_TILE, N_TOTAL) - base, 0)
    nct = (valid + C - 1) // C

    gvs = ((gv0a, gv0b), (gv1a, gv1b))
    gss = ((gs0a, gs0b), (gs1a, gs1b))
    vbs = (vb0, vb1)
    ctxs = (ctx0, ctx1)
    auxs = (aux0, aux1)
    semgs = (semg0, semg1)
    semos = (semo0, semo1)
    semvs = (semv0, semv1)

    full = valid == ROWS_PER_TILE

    @pl.when(full)
    def _stage_full():
        pltpu.sync_copy(idx_hbm.at[pl.ds(base * M, ROWS_PER_TILE * M)], idx_all)
        pltpu.sync_copy(lg_hbm.at[:, pl.ds(base, ROWS_PER_TILE)], lgbuf)

    @pl.when(jnp.logical_not(full))
    def _stage_tail():
        pltpu.sync_copy(idx_hbm.at[pl.ds(base * M, TAIL * M)],
                        idx_all.at[pl.ds(0, TAIL * M)])
        pltpu.sync_copy(lg_hbm.at[:, pl.ds(base, TAIL)],
                        lgbuf.at[:, pl.ds(0, TAIL)])

    npairs = (nct + 1) // 2

    def issue_gather(g, b):
        off = g * C * M
        lo = idx_all.at[pl.ds(off, HALF)]
        hi = idx_all.at[pl.ds(off + HALF, HALF)]
        pltpu.async_copy(gv_hbm.at[lo], gvs[b][0], semgs[b])
        pltpu.async_copy(gv_hbm.at[hi], gvs[b][1], semgs[b])
        pltpu.async_copy(gs_hbm.at[lo], gss[b][0], semgs[b])
        pltpu.async_copy(gs_hbm.at[hi], gss[b][1], semgs[b])

    def wait_gather(g, b):
        off = g * C * M
        lo = idx_all.at[pl.ds(off, HALF)]
        hi = idx_all.at[pl.ds(off + HALF, HALF)]
        pltpu.make_async_copy(gv_hbm.at[lo], gvs[b][0], semgs[b]).wait()
        pltpu.make_async_copy(gv_hbm.at[hi], gvs[b][1], semgs[b]).wait()
        pltpu.make_async_copy(gs_hbm.at[lo], gss[b][0], semgs[b]).wait()
        pltpu.make_async_copy(gs_hbm.at[hi], gss[b][1], semgs[b]).wait()

    def issue_ve(p, pb):
        pltpu.async_copy(ve_hbm.at[:, pl.ds(base + p * 2 * C, 2 * C)],
                         vbs[pb], semvs[pb])

    def wait_ve(p, pb):
        pltpu.make_async_copy(ve_hbm.at[:, pl.ds(base + p * 2 * C, 2 * C)],
                              vbs[pb], semvs[pb]).wait()

    def wait_out(b):
        pltpu.make_async_copy(ctxs[b], out_hbm.at[pl.ds(base, C)], semos[b]).wait()
        pltpu.make_async_copy(auxs[b], aux_hbm.at[pl.ds(base, C)], semos[b]).wait()

    def compute(g, b, pb):
        vb = vbs[pb]
        ctx_v = ctxs[b]
        i16 = lax.iota(jnp.int32, 16)
        for half in (0, 1):
            gv = gvs[b][half]
            gs = gss[b][half]

            def row_body(r4, carry):
                r = half * (C // 2) + r4
                row0 = r4 * M
                lrow = g * C + r
                rc = jnp.full((16,), b * C + r, jnp.int32)
                col0 = jnp.full((16,), 0, jnp.int32)
                lcol = jnp.full((16,), lrow, jnp.int32)
                kw_lo = plsc.load_gather(gs, [i16 + row0, col0])
                kw_hi = plsc.load_gather(gs, [i16 + row0 + 16, col0])
                t_lo = plsc.load_gather(lgbuf, [i16, lcol]) + kw_lo
                t_hi = plsc.load_gather(lgbuf, [i16 + 16, lcol]) + kw_hi
                mx = jnp.maximum(jnp.max(t_lo), jnp.max(t_hi))
                e_lo = jnp.exp(t_lo - mx)
                e_hi = jnp.exp(t_hi - mx)
                sv = jnp.broadcast_to(jnp.sum(e_lo) + jnp.sum(e_hi), (16,))
                a_lo = e_lo / sv
                a_hi = e_hi / sv
                accs = [jnp.zeros((16,), jnp.float32) for _ in range(8)]
                acc_s = jnp.zeros((16,), jnp.float32)
                acc_e = jnp.zeros((16,), jnp.float32)
                for mm in range(M):
                    a = a_lo[mm] if mm < 16 else a_hi[mm - 16]
                    grow = row0 + mm
                    acc_s = acc_s + a * gs[grow, pl.ds(0, 16)]
                    for kk in range(4):
                        vb32 = gv[grow, pl.ds(kk * 32, 32)]
                        vlo, vhi = plsc.unpack(
                            vb32, format=plsc.PackFormat.INTERLEAVED)
                        accs[2 * kk] = accs[2 * kk] + a * vlo
                        accs[2 * kk + 1] = accs[2 * kk + 1] + a * vhi
                    acc_e = acc_e + a * plsc.load_gather(vb, [i16 + mm * DE, rc])
                for kk in range(8):
                    ctx_v[r, pl.ds(kk * 16, 16)] = accs[kk]
                ctx_v[r, pl.ds(D, 16)] = acc_e
                auxs[b][r, pl.ds(0, 16)] = acc_s
                return carry

            lax.fori_loop(0, C // 2, row_body, 0)

    issue_ve(0, 0)
    issue_gather(0, 0)

    def outer(g4, carry):
        for pb in (0, 1):
            p = g4 * 2 + pb

            @pl.when(p < npairs)
            def _pair():
                @pl.when(p + 1 < npairs)
                def _prefetch_ve():
                    issue_ve(p + 1, 1 - pb)

                wait_ve(p, pb)
                for b in (0, 1):
                    g = 2 * p + b

                    @pl.when(g < nct)
                    def _process():
                        @pl.when(g + 1 < nct)
                        def _prefetch():
                            issue_gather(g + 1, 1 - b)

                        wait_gather(g, b)
                        rb = base + g * C

                        @pl.when(g >= 2)
                        def _drain_prev_out():
                            wait_out(b)

                        compute(g, b, pb)
                        pltpu.async_copy(ctxs[b], out_hbm.at[pl.ds(rb, C)],
                                         semos[b])
                        pltpu.async_copy(auxs[b], aux_hbm.at[pl.ds(rb, C)],
                                         semos[b])
        return carry

    lax.fori_loop(0, (npairs + 1) // 2, outer, 0)
    wait_out(0)
    wait_out(1)


_sc_call_cache = []


def _sc_call(idx_flat, lg_t, ve_t, Gv, Gs):
    if not _sc_call_cache:
        _sc_call_cache.append(pl.kernel(
            _sc_body,
            out_type=(jax.ShapeDtypeStruct((N_TOTAL, CW), jnp.float32),
                      jax.ShapeDtypeStruct((N_TOTAL, XW), jnp.float32)),
            mesh=plsc.VectorSubcoreMesh(core_axis_name="c", subcore_axis_name="s"),
            scratch_types=[
                pltpu.VMEM((ROWS_PER_TILE * M,), jnp.int32),
                pltpu.VMEM((M, ROWS_PER_TILE), jnp.float32),
                pltpu.VMEM((HALF, D), jnp.bfloat16),
                pltpu.VMEM((HALF, D), jnp.bfloat16),
                pltpu.VMEM((HALF, D), jnp.bfloat16),
                pltpu.VMEM((HALF, D), jnp.bfloat16),
                pltpu.VMEM((HALF, SW), jnp.float32),
                pltpu.VMEM((HALF, SW), jnp.float32),
                pltpu.VMEM((HALF, SW), jnp.float32),
                pltpu.VMEM((HALF, SW), jnp.float32),
                pltpu.VMEM((M * DE, 2 * C), jnp.float32),
                pltpu.VMEM((M * DE, 2 * C), jnp.float32),
                pltpu.VMEM((C, CW), jnp.float32),
                pltpu.VMEM((C, CW), jnp.float32),
                pltpu.VMEM((C, XW), jnp.float32),
                pltpu.VMEM((C, XW), jnp.float32),
                pltpu.SemaphoreType.DMA,
                pltpu.SemaphoreType.DMA,
                pltpu.SemaphoreType.DMA,
                pltpu.SemaphoreType.DMA,
                pltpu.SemaphoreType.DMA,
                pltpu.SemaphoreType.DMA,
            ],
            compiler_params=pltpu.CompilerParams(
                use_tc_tiling_on_sc=False, needs_layout_passes=False),
        ))
    return _sc_call_cache[0](idx_flat, lg_t, ve_t, Gv, Gs)


def _mlp_kernel(ctx_ref, tok_ref, fc1T_ref, fc1b_ref, lng_ref, lnb_ref,
                fc2T_ref, fc2b_ref, out_ref):
    hin = ctx_ref[...]
    h = jnp.dot(hin, fc1T_ref[...], preferred_element_type=jnp.float32)
    h = h + fc1b_ref[...]
    h = 0.5 * h * (1.0 + lax.erf(h * 0.7071067811865476))
    mu = jnp.mean(h, axis=-1, keepdims=True)
    var = jnp.mean((h - mu) ** 2, axis=-1, keepdims=True)
    h = (h - mu) * lax.rsqrt(var + 1e-5) * lng_ref[...] + lnb_ref[...]
    h = jnp.dot(h, fc2T_ref[...], preferred_element_type=jnp.float32)
    h = h + fc2b_ref[...]
    out_ref[...] = h + tok_ref[...]


def _row_spec(block, width):
    return pl.BlockSpec((block, width), lambda i: (i, 0))


def _full_spec(shape):
    return pl.BlockSpec(shape, lambda i: tuple(0 for _ in shape))


def kernel(token_embs, geo_feats, edge_feats, ln1_g, ln1_b, W_qkv, b_qkv,
           ln_e_g, ln_e_b, W_qkv_e, b_qkv_e, w_attn, w_eattn,
           W_gate_w, W_gate_b, fc1_W, fc1_b, mlp_ln_g, mlp_ln_b, fc2_W, fc2_b,
           neighbor_indices, batch_idx, neighbor_masks):
    f32 = jnp.float32
    N = token_embs.shape[0]

    idx_flat = neighbor_indices.astype(jnp.int32).reshape(N * M)

    WkvT = W_qkv[D:].T
    bkv = b_qkv[D:].reshape(1, 2 * D)
    wattn2 = w_attn.reshape(1, D)

    Gv, Gs = pl.pallas_call(
        _tok_kernel,
        grid=(NB1,),
        in_specs=[
            _row_spec(B1, D),
            _row_spec(B1, 3),
            _full_spec((1, D)),
            _full_spec((1, D)),
            _full_spec((D, 2 * D)),
            _full_spec((1, 2 * D)),
            _full_spec((1, D)),
            _full_spec((D, D)),
        ],
        out_specs=[_row_spec(B1, D), _row_spec(B1, SW)],
        out_shape=[jax.ShapeDtypeStruct((N, D), jnp.bfloat16),
                   jax.ShapeDtypeStruct((N, SW), f32)],
    )(token_embs, geo_feats, ln1_g.reshape(1, D), ln1_b.reshape(1, D),
      WkvT, bkv, wattn2, jnp.asarray(_PERM_MAT))

    et = edge_feats.transpose(1, 2, 0).reshape(M * DE, N)
    Wv = W_qkv_e[DE:]
    bve = b_qkv_e[DE:].reshape(DE, 1)
    wcomb = (w_eattn @ W_qkv_e[:DE]).reshape(1, DE)
    c0 = jnp.dot(b_qkv_e[:DE], w_eattn).reshape(1, 1)
    gcol = ln_e_g.reshape(DE, 1)
    bcol = ln_e_b.reshape(DE, 1)

    BR = MPG * DE
    ve_t, lg_t = pl.pallas_call(
        _edge_kernel,
        grid=(M // MPG,),
        in_specs=[
            pl.BlockSpec((BR, N), lambda i: (i, 0)),
            _full_spec((DE, DE)),
            _full_spec((DE, 1)),
            _full_spec((1, DE)),
            _full_spec((1, 1)),
            _full_spec((DE, 1)),
            _full_spec((DE, 1)),
        ],
        out_specs=[pl.BlockSpec((BR, N), lambda i: (i, 0)),
                   pl.BlockSpec((M, N), lambda i: (0, 0))],
        out_shape=[
            jax.ShapeDtypeStruct((M * DE, N), f32),
            jax.ShapeDtypeStruct((M, N), f32),
        ],
    )(et, Wv, bve, wcomb, c0, gcol, bcol)

    ctx, aux = _sc_call(idx_flat, lg_t, ve_t, Gv, Gs)

    out = pl.pallas_call(
        _mlp_kernel,
        grid=(NB1,),
        in_specs=[
            _row_spec(B1, CW),
            _row_spec(B1, D),
            _full_spec((D + DE, D)),
            _full_spec((1, D)),
            _full_spec((1, D)),
            _full_spec((1, D)),
            _full_spec((D, D)),
            _full_spec((1, D)),
        ],
        out_specs=_row_spec(B1, D),
        out_shape=jax.ShapeDtypeStruct((N, D), f32),
    )(ctx, token_embs, fc1_W.T, fc1_b.reshape(1, D), mlp_ln_g.reshape(1, D),
      mlp_ln_b.reshape(1, D), fc2_W.T, fc2_b.reshape(1, D))

    geo_context = aux[:, 1:4]
    return out, geo_context
